# baseline (speedup 1.0000x reference)
import functools

import jax
import jax.numpy as jnp
from jax import lax
from jax.experimental import pallas as pl
from jax.experimental.pallas import tpu as pltpu
from jax.experimental.pallas import tpu_sc as plsc

B = 16384
L = 20
D = 32
V_PROD = 1000000
NC = 2
NS = 16
NW = NC * NS
SPW = B // NW
CH = 64
NCH = SPW // CH
RPC = CH * L
KCH = RPC // 128


def _sc_body(p1, c1, g1, l1, Wp, Wc, Wg, Wl, out,
             idxp, idxc, idxg, lidx, rows_a, rows_b, sum_v,
             ga, gb, ws):
    cid = lax.axis_index("c")
    sid = lax.axis_index("s")
    wid = sid * NC + cid
    base = wid * SPW

    pltpu.sync_copy(p1.at[pl.ds(base * L, SPW * L)], idxp)
    pltpu.sync_copy(c1.at[pl.ds(base * L, SPW * L)], idxc)
    pltpu.sync_copy(g1.at[pl.ds(base * L, SPW * L)], idxg)
    pltpu.sync_copy(l1.at[pl.ds(base, SPW)], lidx)

    def fire(tbl, idx_v, ch, buf, sem):
        for k in range(KCH):
            pltpu.async_copy(
                tbl.at[idx_v.at[pl.ds(ch * RPC + k * 128, 128)]],
                buf.at[pl.ds(k * 128, 128)], sem)

    def fire_l(ch, buf, sem):
        pltpu.async_copy(Wl.at[lidx.at[pl.ds(ch * CH, CH)]],
                         buf.at[pl.ds(0, CH)], sem)

    def drain(buf, sem):
        pltpu.make_async_copy(Wp.at[pl.ds(0, RPC)], buf, sem).wait()

    def drain_l(buf, sem):
        pltpu.make_async_copy(Wp.at[pl.ds(0, CH)],
                              buf.at[pl.ds(0, CH)], sem).wait()

    def drain_w():
        pltpu.make_async_copy(Wp.at[pl.ds(0, 4 * CH)], sum_v, ws).wait()

    def compute_table(buf, col0):
        def sbody(smp, c):
            r0 = smp * L
            acc0 = jnp.zeros((16,), jnp.float32)
            acc1 = jnp.zeros((16,), jnp.float32)
            for j in range(L):
                acc0 = acc0 + buf[r0 + j, pl.ds(0, 16)]
                acc1 = acc1 + buf[r0 + j, pl.ds(16, 16)]
            sum_v[smp, pl.ds(col0, 16)] = acc0
            sum_v[smp, pl.ds(col0 + 16, 16)] = acc1
            return c

        lax.fori_loop(0, CH, sbody, 0)

    def compute_l(buf):
        def lbody(smp, c):
            sum_v[smp, pl.ds(96, 16)] = buf[smp, pl.ds(0, 16)]
            sum_v[smp, pl.ds(112, 16)] = buf[smp, pl.ds(16, 16)]
            return c

        lax.fori_loop(0, CH, lbody, 0)

    fire(Wp, idxp, 0, rows_a, ga)

    def do_chunk(ch, carry):
        drain(rows_a, ga)
        fire(Wc, idxc, ch, rows_b, gb)

        @pl.when(ch > 0)
        def _():
            drain_w()

        compute_table(rows_a, 0)

        drain(rows_b, gb)
        fire(Wg, idxg, ch, rows_a, ga)
        compute_table(rows_b, 32)

        drain(rows_a, ga)
        fire_l(ch, rows_b, gb)
        compute_table(rows_a, 64)

        drain_l(rows_b, gb)
        fire(Wp, idxp, jnp.minimum(ch + 1, NCH - 1), rows_a, ga)
        compute_l(rows_b)

        pltpu.async_copy(sum_v, out.at[pl.ds(base + ch * CH, CH)], ws)
        return carry

    lax.fori_loop(0, NCH, do_chunk, 0)
    drain(rows_a, ga)
    drain_w()


@jax.jit
def _sc_call(p1, c1, g1, l1, Wp, Wc, Wg, Wl):
    mesh = plsc.VectorSubcoreMesh(core_axis_name="c", subcore_axis_name="s")
    return pl.kernel(
        _sc_body,
        out_type=jax.ShapeDtypeStruct((B, 4 * D), jnp.float32),
        mesh=mesh,
        compiler_params=pltpu.CompilerParams(use_tc_tiling_on_sc=False),
        scratch_types=[
            pltpu.VMEM((SPW * L,), jnp.int32),
            pltpu.VMEM((SPW * L,), jnp.int32),
            pltpu.VMEM((SPW * L,), jnp.int32),
            pltpu.VMEM((SPW,), jnp.int32),
            pltpu.VMEM((RPC, D), jnp.float32),
            pltpu.VMEM((RPC, D), jnp.float32),
            pltpu.VMEM((CH, 4 * D), jnp.float32),
            pltpu.SemaphoreType.DMA,
            pltpu.SemaphoreType.DMA,
            pltpu.SemaphoreType.DMA,
        ],
    )(p1, c1, g1, l1, Wp, Wc, Wg, Wl)


NSTRIP = 7813
SBASE = NSTRIP // NW
SEXTRA = NSTRIP - SBASE * NW
NSS = 8


RL_BS = 4
RL_COLS = RL_BS * 128
NBLK = SBASE // RL_BS


TP = RL_COLS + 1


def _rl_body(wpt, wct, wgt, wlt, out_p, out_c, out_g, out_l,
             tile_a, tile_b, row_a, row_b, ga, gb, wa, wb):
    cid = lax.axis_index("c")
    sid = lax.axis_index("s")
    wid = sid * NC + cid
    i32 = jnp.int32
    iota = lax.iota(i32, 16)
    drow = [d0 + iota for d0 in (0, 16)]
    s0 = wid * SBASE + jnp.minimum(wid, SEXTRA)
    c0 = s0 * 128

    def fire_gather(blk, tile, sem):
        pltpu.async_copy(
            wpt.at[pl.ds(0, D), pl.ds(c0 + blk * RL_COLS, RL_COLS)],
            tile.at[pl.ds(0, D), pl.ds(0, RL_COLS)], sem)

    def drain_gather(tile, sem):
        pltpu.make_async_copy(
            wpt.at[pl.ds(0, D), pl.ds(0, RL_COLS)],
            tile.at[pl.ds(0, D), pl.ds(0, RL_COLS)], sem).wait()

    def fire_write(row, blk, sem):
        pltpu.async_copy(
            row, out_p.at[pl.ds((s0 + blk * RL_BS) * D, RL_BS * D)], sem)

    def drain_write(row, sem):
        pltpu.make_async_copy(out_p.at[pl.ds(0, RL_BS * D)], row, sem).wait()

    def transpose_rows(tile, row, ngroups):
        def ibody(i0, carry):
            vss = [plsc.load_gather(
                       tile, [drow[h], jnp.full((16,), i0 * 8 + i, i32)])
                   for i in range(8) for h in range(2)]
            for i in range(8):
                for h in range(2):
                    row[(i0 * 8 + i) // 4,
                        pl.ds(((i0 * 8 + i) % 4) * D + 16 * h, 16)] = \
                        vss[i * 2 + h]
            return carry

        lax.fori_loop(0, ngroups, ibody, 0)

    def transpose_block(tile, row):
        transpose_rows(tile, row, RL_COLS // 8)

    fire_gather(0, tile_a, ga)

    def loop(t, carry):
        drain_gather(tile_a, ga)
        fire_gather(2 * t + 1, tile_b, gb)

        @pl.when(t > 0)
        def _():
            drain_write(row_a, wa)

        transpose_block(tile_a, row_a)
        fire_write(row_a, 2 * t, wa)

        drain_gather(tile_b, gb)
        fire_gather(2 * t + 2, tile_a, ga)

        @pl.when(t > 0)
        def _():
            drain_write(row_b, wb)

        transpose_block(tile_b, row_b)
        fire_write(row_b, 2 * t + 1, wb)
        return carry

    lax.fori_loop(0, (NBLK - 1) // 2, loop, 0)
    drain_gather(tile_a, ga)
    drain_write(row_a, wa)
    transpose_block(tile_a, row_a)
    fire_write(row_a, NBLK - 1, wa)
    drain_write(row_a, wa)
    drain_write(row_b, wb)

    def sync_strip(src, dst, strip, dst_row0):
        pltpu.async_copy(
            src.at[pl.ds(0, D), pl.ds(strip * 128, 128)],
            tile_a.at[pl.ds(0, D), pl.ds(0, 128)], ga).wait()
        transpose_rows(tile_a, row_a, 16)
        pltpu.async_copy(row_a.at[pl.ds(0, D)],
                         dst.at[pl.ds(dst_row0, D)], wa).wait()

    @pl.when(wid < SEXTRA)
    def _():
        sync_strip(wpt, out_p, s0 + SBASE, (s0 + SBASE) * D)

    @pl.when(jnp.logical_and(wid >= 8, wid < 8 + 3 * NSS))
    def _():
        q = (wid - 8) % NSS
        tsel = (wid - 8) // NSS

        @pl.when(tsel == 0)
        def _():
            sync_strip(wct, out_c, q, q * D)

        @pl.when(tsel == 1)
        def _():
            sync_strip(wgt, out_g, q, q * D)

        @pl.when(tsel == 2)
        def _():
            sync_strip(wlt, out_l, q, q * D)


@jax.jit
def _relayout_call(wpt, wct, wgt, wlt):
    mesh = plsc.VectorSubcoreMesh(core_axis_name="c", subcore_axis_name="s")
    return pl.kernel(
        _rl_body,
        out_type=(
            jax.ShapeDtypeStruct((NSTRIP * D, 128), jnp.float32),
            jax.ShapeDtypeStruct((NSS * D, 128), jnp.float32),
            jax.ShapeDtypeStruct((NSS * D, 128), jnp.float32),
            jax.ShapeDtypeStruct((NSS * D, 128), jnp.float32),
        ),
        mesh=mesh,
        compiler_params=pltpu.CompilerParams(use_tc_tiling_on_sc=True,
                                             needs_layout_passes=False),
        scratch_types=[
            pltpu.VMEM((D, TP), jnp.float32),
            pltpu.VMEM((D, TP), jnp.float32),
            pltpu.VMEM((RL_BS * D, 128), jnp.float32),
            pltpu.VMEM((RL_BS * D, 128), jnp.float32),
            pltpu.SemaphoreType.DMA,
            pltpu.SemaphoreType.DMA,
            pltpu.SemaphoreType.DMA,
            pltpu.SemaphoreType.DMA,
        ],
    )(wpt, wct, wgt, wlt)


BLK = 2048


def _tc_body(ctx_ref, p_ref, c_ref, g_ref, r_ref,
             W1_ref, b1_ref, W2_ref, b2_ref, W3_ref, b3_ref,
             out_ref, wpop_ref, wvote_ref):
    f32 = jnp.float32

    def den(iref):
        m = (iref[...] != 0).astype(f32)
        cnt = jnp.maximum(jnp.sum(m, axis=1, keepdims=True), 1.0)
        return jnp.broadcast_to(cnt, (BLK, D))

    denom = jnp.concatenate(
        [den(p_ref), den(c_ref), den(g_ref), jnp.ones((BLK, D), f32)], axis=1)
    ctx = ctx_ref[...] / denom
    h = jnp.maximum(jnp.dot(ctx, W1_ref[...], preferred_element_type=f32)
                    + b1_ref[...], 0.0)
    h = jnp.maximum(jnp.dot(h, W2_ref[...], preferred_element_type=f32)
                    + b2_ref[...], 0.0)
    prm = jnp.dot(h, W3_ref[...], preferred_element_type=f32) + b3_ref[...]
    w_pop = prm[:, 0:1]
    w_vote = prm[:, 1:2]
    b_pop = prm[:, 2:3]
    b_vote = prm[:, 3:4]
    rr = r_ref[...]
    out_ref[...] = jnp.concatenate(
        [w_pop * rr + b_pop, w_vote * rr + b_vote], axis=1)
    wpop_ref[...] = w_pop
    wvote_ref[...] = w_vote


@jax.jit
def _tc_call(ctx, p, c, g, r, W1, b1, W2, b2, W3, b3):
    grid = (B // BLK,)
    bs_row = lambda width: pl.BlockSpec((BLK, width), lambda i: (i, 0))
    bs_full = lambda a, b: pl.BlockSpec((a, b), lambda i: (0, 0))
    return pl.pallas_call(
        _tc_body,
        grid=grid,
        in_specs=[
            bs_row(4 * D), bs_row(L), bs_row(L), bs_row(L), bs_row(1),
            bs_full(4 * D, 16), bs_full(1, 16),
            bs_full(16, 16), bs_full(1, 16),
            bs_full(16, 4), bs_full(1, 4),
        ],
        out_specs=[bs_row(2), bs_row(1), bs_row(1)],
        out_shape=[
            jax.ShapeDtypeStruct((B, 2), jnp.float32),
            jax.ShapeDtypeStruct((B, 1), jnp.float32),
            jax.ShapeDtypeStruct((B, 1), jnp.float32),
        ],
    )(ctx, p, c, g, r, W1, b1, W2, b2, W3, b3)


def kernel(r, p, c, g, l, W_prod, W_country, W_genre, W_lang,
           W1, b1, W2, b2, W3, b3):
    i32 = jnp.int32
    p32 = p.astype(i32)
    c32 = c.astype(i32)
    g32 = g.astype(i32)
    l32 = l.astype(i32)
    wq, wcq, wgq, wlq = _relayout_call(
        W_prod.T, W_country.T, W_genre.T, W_lang.T)
    ctx = _sc_call(p32.reshape(-1), c32.reshape(-1), g32.reshape(-1), l32,
                   wq.reshape(-1, D), wcq.reshape(-1, D),
                   wgq.reshape(-1, D), wlq.reshape(-1, D))
    out, w_pop, w_vote = _tc_call(
        ctx, p32, c32, g32, r,
        W1, b1.reshape(1, 16), W2, b2.reshape(1, 16), W3, b3.reshape(1, 4))
    return out, w_pop, w_vote


# speedup vs baseline: 1.3953x; 1.3953x over previous
import functools

import jax
import jax.numpy as jnp
from jax import lax
from jax.experimental import pallas as pl
from jax.experimental.pallas import tpu as pltpu
from jax.experimental.pallas import tpu_sc as plsc

B = 16384
L = 20
D = 32
V_PROD = 1000000
NC = 2
NS = 16
NW = NC * NS
SPW = B // NW
CH = 64
NCH = SPW // CH
RPC = CH * L
KCH = RPC // 128


def _sc_body(p1, c1, g1, l1, Wp, Wc, Wg, Wl, out,
             idxp, idxc, idxg, lidx, rows_a, rows_b, sum_v,
             ga, gb, ws):
    cid = lax.axis_index("c")
    sid = lax.axis_index("s")
    wid = sid * NC + cid
    base = wid * SPW

    pltpu.sync_copy(p1.at[pl.ds(base * L, SPW * L)], idxp)
    pltpu.sync_copy(c1.at[pl.ds(base * L, SPW * L)], idxc)
    pltpu.sync_copy(g1.at[pl.ds(base * L, SPW * L)], idxg)
    pltpu.sync_copy(l1.at[pl.ds(base, SPW)], lidx)

    def fire(tbl, idx_v, ch, buf, sem):
        for k in range(KCH):
            pltpu.async_copy(
                tbl.at[idx_v.at[pl.ds(ch * RPC + k * 128, 128)]],
                buf.at[pl.ds(k * 128, 128)], sem)

    def fire_l(ch, buf, sem):
        pltpu.async_copy(Wl.at[lidx.at[pl.ds(ch * CH, CH)]],
                         buf.at[pl.ds(0, CH)], sem)

    def drain(buf, sem):
        pltpu.make_async_copy(Wp.at[pl.ds(0, RPC)], buf, sem).wait()

    def drain_l(buf, sem):
        pltpu.make_async_copy(Wp.at[pl.ds(0, CH)],
                              buf.at[pl.ds(0, CH)], sem).wait()

    def drain_w():
        pltpu.make_async_copy(Wp.at[pl.ds(0, 4 * CH)], sum_v, ws).wait()

    def compute_table(buf, col0):
        def sbody(smp, c):
            r0 = smp * L
            acc0 = jnp.zeros((16,), jnp.float32)
            acc1 = jnp.zeros((16,), jnp.float32)
            for j in range(L):
                acc0 = acc0 + buf[r0 + j, pl.ds(0, 16)]
                acc1 = acc1 + buf[r0 + j, pl.ds(16, 16)]
            sum_v[smp, pl.ds(col0, 16)] = acc0
            sum_v[smp, pl.ds(col0 + 16, 16)] = acc1
            return c

        lax.fori_loop(0, CH, sbody, 0)

    def compute_l(buf):
        def lbody(smp, c):
            sum_v[smp, pl.ds(96, 16)] = buf[smp, pl.ds(0, 16)]
            sum_v[smp, pl.ds(112, 16)] = buf[smp, pl.ds(16, 16)]
            return c

        lax.fori_loop(0, CH, lbody, 0)

    fire(Wp, idxp, 0, rows_a, ga)

    def do_chunk(ch, carry):
        drain(rows_a, ga)
        fire(Wc, idxc, ch, rows_b, gb)

        @pl.when(ch > 0)
        def _():
            drain_w()

        compute_table(rows_a, 0)

        drain(rows_b, gb)
        fire(Wg, idxg, ch, rows_a, ga)
        compute_table(rows_b, 32)

        drain(rows_a, ga)
        fire_l(ch, rows_b, gb)
        compute_table(rows_a, 64)

        drain_l(rows_b, gb)
        fire(Wp, idxp, jnp.minimum(ch + 1, NCH - 1), rows_a, ga)
        compute_l(rows_b)

        pltpu.async_copy(sum_v, out.at[pl.ds(base + ch * CH, CH)], ws)
        return carry

    lax.fori_loop(0, NCH, do_chunk, 0)
    drain(rows_a, ga)
    drain_w()


@jax.jit
def _sc_call(p1, c1, g1, l1, Wp, Wc, Wg, Wl):
    mesh = plsc.VectorSubcoreMesh(core_axis_name="c", subcore_axis_name="s")
    return pl.kernel(
        _sc_body,
        out_type=jax.ShapeDtypeStruct((B, 4 * D), jnp.float32),
        mesh=mesh,
        compiler_params=pltpu.CompilerParams(use_tc_tiling_on_sc=False),
        scratch_types=[
            pltpu.VMEM((SPW * L,), jnp.int32),
            pltpu.VMEM((SPW * L,), jnp.int32),
            pltpu.VMEM((SPW * L,), jnp.int32),
            pltpu.VMEM((SPW,), jnp.int32),
            pltpu.VMEM((RPC, D), jnp.float32),
            pltpu.VMEM((RPC, D), jnp.float32),
            pltpu.VMEM((CH, 4 * D), jnp.float32),
            pltpu.SemaphoreType.DMA,
            pltpu.SemaphoreType.DMA,
            pltpu.SemaphoreType.DMA,
        ],
    )(p1, c1, g1, l1, Wp, Wc, Wg, Wl)


NSTRIP = 7813
SBASE = NSTRIP // NW
SEXTRA = NSTRIP - SBASE * NW
NSS = 8


RL_BS = 4
RL_COLS = RL_BS * 128
NBLK = SBASE // RL_BS


PITCH = D + 1


def _rl_body(wpt, wct, wgt, wlt, out_p, out_c, out_g, out_l,
             tile_a, tile_b, p33_a, p33_b, row_a, row_b, ga, gb, wa, wb):
    cid = lax.axis_index("c")
    sid = lax.axis_index("s")
    wid = sid * NC + cid
    i32 = jnp.int32
    iota = lax.iota(i32, 16)
    bm33 = [(m0 + iota) * PITCH for m0 in range(0, 128, 16)]
    s0 = wid * SBASE + jnp.minimum(wid, SEXTRA)
    c0 = s0 * 128

    def fire_gather(blk, tile, sem):
        pltpu.async_copy(
            wpt.at[pl.ds(0, D), pl.ds(c0 + blk * RL_COLS, RL_COLS)], tile, sem)

    def drain_gather(tile, sem):
        pltpu.make_async_copy(
            wpt.at[pl.ds(0, D), pl.ds(0, RL_COLS)], tile, sem).wait()

    def fire_write(row, blk, sem):
        pltpu.async_copy(
            row, out_p.at[pl.ds((s0 + blk * RL_BS) * D, RL_BS * D)], sem)

    def drain_write(row, sem):
        pltpu.make_async_copy(out_p.at[pl.ds(0, RL_BS * D)], row, sem).wait()

    def transpose_strip(tile, p33, row, s, scol):
        soff = s * 128 * PITCH
        for m in range(8):
            bms = bm33[m] + soff
            for d0 in range(0, D, 8):
                vs = [tile[d0 + u, pl.ds(scol + m * 16, 16)]
                      for u in range(8)]
                for u in range(8):
                    plsc.store_scatter(p33, [bms + (d0 + u)], vs[u])
        for rr in range(0, 128, 4):
            vs = [p33[pl.ds(soff + (rr + q) * PITCH + 16 * h, 16)]
                  for q in range(4) for h in range(2)]
            for q in range(4):
                for h in range(2):
                    row[s * D + rr // 4, pl.ds(q * D + 16 * h, 16)] = \
                        vs[q * 2 + h]

    def transpose_block(tile, p33, row):
        def tstrip(s, carry):
            transpose_strip(tile, p33, row, s, s * 128)
            return carry

        lax.fori_loop(0, RL_BS, tstrip, 0)

    fire_gather(0, tile_a, ga)

    def loop(t, carry):
        drain_gather(tile_a, ga)
        fire_gather(2 * t + 1, tile_b, gb)

        @pl.when(t > 0)
        def _():
            drain_write(row_a, wa)

        transpose_block(tile_a, p33_a, row_a)
        fire_write(row_a, 2 * t, wa)

        drain_gather(tile_b, gb)
        fire_gather(2 * t + 2, tile_a, ga)

        @pl.when(t > 0)
        def _():
            drain_write(row_b, wb)

        transpose_block(tile_b, p33_b, row_b)
        fire_write(row_b, 2 * t + 1, wb)
        return carry

    lax.fori_loop(0, (NBLK - 1) // 2, loop, 0)
    drain_gather(tile_a, ga)
    drain_write(row_a, wa)
    transpose_block(tile_a, p33_a, row_a)
    fire_write(row_a, NBLK - 1, wa)
    drain_write(row_a, wa)
    drain_write(row_b, wb)

    def sync_strip(src, dst, strip, dst_row0):
        pltpu.async_copy(
            src.at[pl.ds(0, D), pl.ds(strip * 128, 128)],
            tile_a.at[:, pl.ds(0, 128)], ga).wait()
        transpose_strip(tile_a, p33_a, row_a, 0, 0)
        pltpu.async_copy(row_a.at[pl.ds(0, D)],
                         dst.at[pl.ds(dst_row0, D)], wa).wait()

    @pl.when(wid < SEXTRA)
    def _():
        sync_strip(wpt, out_p, s0 + SBASE, (s0 + SBASE) * D)

    @pl.when(jnp.logical_and(wid >= 8, wid < 8 + 3 * NSS))
    def _():
        q = (wid - 8) % NSS
        tsel = (wid - 8) // NSS

        @pl.when(tsel == 0)
        def _():
            sync_strip(wct, out_c, q, q * D)

        @pl.when(tsel == 1)
        def _():
            sync_strip(wgt, out_g, q, q * D)

        @pl.when(tsel == 2)
        def _():
            sync_strip(wlt, out_l, q, q * D)


@jax.jit
def _relayout_call(wpt, wct, wgt, wlt):
    mesh = plsc.VectorSubcoreMesh(core_axis_name="c", subcore_axis_name="s")
    return pl.kernel(
        _rl_body,
        out_type=(
            jax.ShapeDtypeStruct((NSTRIP * D, 128), jnp.float32),
            jax.ShapeDtypeStruct((NSS * D, 128), jnp.float32),
            jax.ShapeDtypeStruct((NSS * D, 128), jnp.float32),
            jax.ShapeDtypeStruct((NSS * D, 128), jnp.float32),
        ),
        mesh=mesh,
        compiler_params=pltpu.CompilerParams(use_tc_tiling_on_sc=True,
                                             needs_layout_passes=False),
        scratch_types=[
            pltpu.VMEM((D, RL_COLS), jnp.float32),
            pltpu.VMEM((D, RL_COLS), jnp.float32),
            pltpu.VMEM((RL_COLS * PITCH,), jnp.float32),
            pltpu.VMEM((RL_COLS * PITCH,), jnp.float32),
            pltpu.VMEM((RL_BS * D, 128), jnp.float32),
            pltpu.VMEM((RL_BS * D, 128), jnp.float32),
            pltpu.SemaphoreType.DMA,
            pltpu.SemaphoreType.DMA,
            pltpu.SemaphoreType.DMA,
            pltpu.SemaphoreType.DMA,
        ],
    )(wpt, wct, wgt, wlt)


BLK = 2048


def _tc_body(ctx_ref, p_ref, c_ref, g_ref, r_ref,
             W1_ref, b1_ref, W2_ref, b2_ref, W3_ref, b3_ref,
             out_ref, wpop_ref, wvote_ref):
    f32 = jnp.float32

    def den(iref):
        m = (iref[...] != 0).astype(f32)
        cnt = jnp.maximum(jnp.sum(m, axis=1, keepdims=True), 1.0)
        return jnp.broadcast_to(cnt, (BLK, D))

    denom = jnp.concatenate(
        [den(p_ref), den(c_ref), den(g_ref), jnp.ones((BLK, D), f32)], axis=1)
    ctx = ctx_ref[...] / denom
    h = jnp.maximum(jnp.dot(ctx, W1_ref[...], preferred_element_type=f32)
                    + b1_ref[...], 0.0)
    h = jnp.maximum(jnp.dot(h, W2_ref[...], preferred_element_type=f32)
                    + b2_ref[...], 0.0)
    prm = jnp.dot(h, W3_ref[...], preferred_element_type=f32) + b3_ref[...]
    w_pop = prm[:, 0:1]
    w_vote = prm[:, 1:2]
    b_pop = prm[:, 2:3]
    b_vote = prm[:, 3:4]
    rr = r_ref[...]
    out_ref[...] = jnp.concatenate(
        [w_pop * rr + b_pop, w_vote * rr + b_vote], axis=1)
    wpop_ref[...] = w_pop
    wvote_ref[...] = w_vote


@jax.jit
def _tc_call(ctx, p, c, g, r, W1, b1, W2, b2, W3, b3):
    grid = (B // BLK,)
    bs_row = lambda width: pl.BlockSpec((BLK, width), lambda i: (i, 0))
    bs_full = lambda a, b: pl.BlockSpec((a, b), lambda i: (0, 0))
    return pl.pallas_call(
        _tc_body,
        grid=grid,
        in_specs=[
            bs_row(4 * D), bs_row(L), bs_row(L), bs_row(L), bs_row(1),
            bs_full(4 * D, 16), bs_full(1, 16),
            bs_full(16, 16), bs_full(1, 16),
            bs_full(16, 4), bs_full(1, 4),
        ],
        out_specs=[bs_row(2), bs_row(1), bs_row(1)],
        out_shape=[
            jax.ShapeDtypeStruct((B, 2), jnp.float32),
            jax.ShapeDtypeStruct((B, 1), jnp.float32),
            jax.ShapeDtypeStruct((B, 1), jnp.float32),
        ],
    )(ctx, p, c, g, r, W1, b1, W2, b2, W3, b3)


def kernel(r, p, c, g, l, W_prod, W_country, W_genre, W_lang,
           W1, b1, W2, b2, W3, b3):
    i32 = jnp.int32
    p32 = p.astype(i32)
    c32 = c.astype(i32)
    g32 = g.astype(i32)
    l32 = l.astype(i32)
    wq, wcq, wgq, wlq = _relayout_call(
        W_prod.T, W_country.T, W_genre.T, W_lang.T)
    ctx = _sc_call(p32.reshape(-1), c32.reshape(-1), g32.reshape(-1), l32,
                   wq.reshape(-1, D), wcq.reshape(-1, D),
                   wgq.reshape(-1, D), wlq.reshape(-1, D))
    out, w_pop, w_vote = _tc_call(
        ctx, p32, c32, g32, r,
        W1, b1.reshape(1, 16), W2, b2.reshape(1, 16), W3, b3.reshape(1, 4))
    return out, w_pop, w_vote
